# baseline (speedup 1.0000x reference)
import functools

import jax
import jax.numpy as jnp
from jax import lax
from jax.experimental import pallas as pl
from jax.experimental.pallas import tpu as pltpu
from jax.experimental.pallas import tpu_sc as plsc

NC, NS = 2, 16
NW = NC * NS
CH = 128
UNIT = NW * CH
BM = 400
BE = 2048
U = 4


def _sc_mesh():
    return plsc.VectorSubcoreMesh(core_axis_name="c", subcore_axis_name="s",
                                  num_cores=NC, num_subcores=NS)


def _pad_edges(src, dst, n):
    e = src.shape[0]
    ep = -(-e // UNIT) * UNIT
    if ep != e:
        fill = n + (jnp.arange(ep - e, dtype=jnp.int32) % 16)
        src = jnp.concatenate([src, fill])
        dst = jnp.concatenate([dst, fill])
    return src, dst, ep


def _sc_degrees(dst1, dst2, np_, zeros):
    nch1 = dst1.shape[0] // UNIT
    nch2 = dst2.shape[0] // UNIT
    rp = np_ // NS
    ones = jnp.ones((CH, 16), jnp.float32)
    d1r = dst1.reshape(-1, CH)
    d2r = dst2.reshape(-1, CH)

    @functools.partial(
        pl.kernel,
        out_type=(jax.ShapeDtypeStruct((NC * np_, 16), jnp.float32),
                  jax.ShapeDtypeStruct((NC * np_, 16), jnp.float32)),
        mesh=_sc_mesh(),
        compiler_params=pltpu.CompilerParams(use_tc_tiling_on_sc=False),
        scratch_types=[
            pltpu.VMEM((U, CH), jnp.int32),
            pltpu.VMEM((CH, 16), jnp.float32),
            pltpu.VMEM_SHARED((np_, 16), jnp.float32),
        ] + [pltpu.SemaphoreType.DMA] * U,
    )
    def k(d1_hbm, d2_hbm, ones_hbm, z_hbm, o1_hbm, o2_hbm,
          idx_v, ones_v, acc, *sems):
        cid = lax.axis_index("c")
        sid = lax.axis_index("s")
        wid = cid * NS + sid
        row = cid * np_ + sid * rp
        pltpu.sync_copy(ones_hbm, ones_v)

        def one_set(d_hbm, nch, o_hbm):
            pltpu.sync_copy(z_hbm.at[pl.ds(sid * rp, rp)],
                            acc.at[pl.ds(sid * rp, rp)])
            plsc.subcore_barrier()
            base = wid * nch

            @pl.loop(0, nch // U)
            def _(j):
                pltpu.sync_copy(d_hbm.at[pl.ds(base + j * U, U)], idx_v)
                ds_ = [pltpu.async_copy(ones_v, acc.at[idx_v.at[b]], sems[b],
                                        add=True) for b in range(U)]
                for d_ in ds_:
                    d_.wait()

            plsc.subcore_barrier()
            pltpu.sync_copy(acc.at[pl.ds(sid * rp, rp)], o_hbm.at[pl.ds(row, rp)])

        one_set(d1_hbm, nch1, o1_hbm)
        plsc.subcore_barrier()
        one_set(d2_hbm, nch2, o2_hbm)

    return k(d1r, d2r, ones, zeros)


def _sc_segment_sum(g_pad, src, dst, np_, d, zeros, us, linear, chp=CH):
    nch = src.shape[0] // (NW * chp)
    rp = np_ // NS
    s2d = src.reshape(-1, chp)
    d2d = dst.reshape(-1, chp)
    US = us

    @functools.partial(
        pl.kernel,
        out_type=jax.ShapeDtypeStruct((NC * np_, d), jnp.float32),
        mesh=_sc_mesh(),
        compiler_params=(pltpu.CompilerParams(use_tc_tiling_on_sc=False)
                         if linear else None),
        scratch_types=[
            pltpu.VMEM((US, chp), jnp.int32),
            pltpu.VMEM((US, chp), jnp.int32),
        ] + [pltpu.VMEM((chp, d), jnp.float32)] * US + [
            pltpu.VMEM_SHARED((np_, d), jnp.float32),
        ] + [pltpu.SemaphoreType.DMA] * (2 * US),
    )
    def k(g_hbm, s_hbm, d_hbm, z_hbm, o_hbm, sidx, didx, *rest):
        rows = rest[:US]
        acc = rest[US]
        gsem = rest[US + 1:US + 1 + US]
        ssem = rest[US + 1 + US:]
        cid = lax.axis_index("c")
        sid = lax.axis_index("s")
        wid = cid * NS + sid
        pltpu.sync_copy(z_hbm.at[pl.ds(sid * rp, rp)], acc.at[pl.ds(sid * rp, rp)])
        plsc.subcore_barrier()
        base = wid * nch

        @pl.loop(0, nch // US)
        def _(j):
            row0 = base + j * US
            pltpu.sync_copy(s_hbm.at[pl.ds(row0, US)], sidx)
            pltpu.sync_copy(d_hbm.at[pl.ds(row0, US)], didx)
            gd = [pltpu.async_copy(g_hbm.at[sidx.at[b]], rows[b], gsem[b])
                  for b in range(US)]
            sd = []
            for b in range(US):
                gd[b].wait()
                sd.append(pltpu.async_copy(rows[b], acc.at[didx.at[b]],
                                           ssem[b], add=True))
            for d_ in sd:
                d_.wait()

        plsc.subcore_barrier()
        row = cid * np_ + sid * rp
        pltpu.sync_copy(acc.at[pl.ds(sid * rp, rp)], o_hbm.at[pl.ds(row, rp)])

    return k(g_pad, s2d, d2d, zeros)


def _sc_link_logits(x_pad, idx_j, idx_i, d, real_c):
    elp = idx_j.shape[0]
    nch = elp // UNIT
    UL = 2
    nv = real_c // 16
    j2d = idx_j.reshape(-1, CH)
    i2d = idx_i.reshape(-1, CH)

    @functools.partial(
        pl.kernel,
        out_type=jax.ShapeDtypeStruct((elp,), jnp.float32),
        mesh=_sc_mesh(),
        compiler_params=pltpu.CompilerParams(needs_layout_passes=False,
                                             use_tc_tiling_on_sc=False),
        scratch_types=[
            pltpu.VMEM((UL, CH), jnp.int32),
            pltpu.VMEM((UL, CH), jnp.int32),
        ] + [pltpu.VMEM((CH, d), jnp.float32)] * (2 * UL)
          + [pltpu.VMEM((CH,), jnp.float32)] * UL
          + [pltpu.SemaphoreType.DMA] * (3 * UL),
    )
    def k(x_hbm, j_hbm, i_hbm, o_hbm, jidx, iidx, *rest):
        jrows = rest[:UL]
        irows = rest[UL:2 * UL]
        outv = rest[2 * UL:3 * UL]
        sems = rest[3 * UL:]
        cid = lax.axis_index("c")
        sid = lax.axis_index("s")
        wid = cid * NS + sid
        base = wid * nch

        @pl.loop(0, nch // UL)
        def _(jj):
            row0 = base + jj * UL
            pltpu.sync_copy(j_hbm.at[pl.ds(row0, UL)], jidx)
            pltpu.sync_copy(i_hbm.at[pl.ds(row0, UL)], iidx)
            gd = []
            for b in range(UL):
                gd.append(pltpu.async_copy(x_hbm.at[jidx.at[b]], jrows[b],
                                           sems[2 * b]))
                gd.append(pltpu.async_copy(x_hbm.at[iidx.at[b]], irows[b],
                                           sems[2 * b + 1]))
            wd = []
            for b in range(UL):
                gd[2 * b].wait()
                gd[2 * b + 1].wait()

                def dot_group(grp, b=b):
                    lane = lax.iota(jnp.int32, 16)
                    vec = jnp.zeros((16,), jnp.float32)
                    for rr in range(16):
                        r = grp * 16 + rr
                        acc = (jrows[b][r, pl.ds(0, 16)] *
                               irows[b][r, pl.ds(0, 16)])
                        for kk in range(1, nv):
                            acc = acc + (jrows[b][r, pl.ds(16 * kk, 16)] *
                                         irows[b][r, pl.ds(16 * kk, 16)])
                        vec = jnp.where(lane == rr, jnp.sum(acc), vec)
                    outv[b][pl.ds(grp * 16, 16)] = vec

                pl.loop(0, CH // 16)(dot_group)
                wd.append(pltpu.async_copy(outv[b],
                                           o_hbm.at[pl.ds((row0 + b) * CH, CH)],
                                           sems[2 * UL + b]))
            for d_ in wd:
                d_.wait()

    return k(x_pad, j2d, i2d)


def _tc_matmul_scale(x, w, degp):
    n, kdim = x.shape
    m = w.shape[1]

    def body(x_ref, w_ref, d0_ref, d1_ref, o_ref):
        h = jnp.dot(x_ref[...], w_ref[...], preferred_element_type=jnp.float32)
        deg = d0_ref[0, :, 0] + d1_ref[0, :, 0] + 1.0
        o_ref[...] = h * lax.rsqrt(deg)[:, None]

    return pl.pallas_call(
        body,
        grid=(n // BM,),
        in_specs=[pl.BlockSpec((BM, kdim), lambda i: (i, 0)),
                  pl.BlockSpec((kdim, m), lambda i: (0, 0)),
                  pl.BlockSpec((1, BM, 16), lambda i: (0, i, 0)),
                  pl.BlockSpec((1, BM, 16), lambda i: (1, i, 0))],
        out_specs=pl.BlockSpec((BM, m), lambda i: (i, 0)),
        out_shape=jax.ShapeDtypeStruct((n, m), jnp.float32),
    )(x, w, degp, degp)


def _tc_layer2(p1, g1, degp1, b1, w2, degp2):
    n, h = g1.shape
    c = w2.shape[1]
    np_ = degp1.shape[1]

    def body(p0_ref, p1_ref, g_ref, d10_ref, d11_ref, b_ref, w_ref,
             d20_ref, d21_ref, o_ref):
        deg1 = d10_ref[0, :, 0] + d11_ref[0, :, 0] + 1.0
        s = p0_ref[0] + p1_ref[0] + g_ref[...]
        x1 = jnp.maximum(s * lax.rsqrt(deg1)[:, None] + b_ref[...], 0.0)
        h2 = jnp.dot(x1, w_ref[...], preferred_element_type=jnp.float32)
        deg2 = d20_ref[0, :, 0] + d21_ref[0, :, 0] + 1.0
        o_ref[...] = h2 * lax.rsqrt(deg2)[:, None]

    return pl.pallas_call(
        body,
        grid=(n // BM,),
        in_specs=[pl.BlockSpec((1, BM, h), lambda i: (0, i, 0)),
                  pl.BlockSpec((1, BM, h), lambda i: (1, i, 0)),
                  pl.BlockSpec((BM, h), lambda i: (i, 0)),
                  pl.BlockSpec((1, BM, 16), lambda i: (0, i, 0)),
                  pl.BlockSpec((1, BM, 16), lambda i: (1, i, 0)),
                  pl.BlockSpec((1, h), lambda i: (0, 0)),
                  pl.BlockSpec((h, c), lambda i: (0, 0)),
                  pl.BlockSpec((1, BM, 16), lambda i: (0, i, 0)),
                  pl.BlockSpec((1, BM, 16), lambda i: (1, i, 0))],
        out_specs=pl.BlockSpec((BM, c), lambda i: (i, 0)),
        out_shape=jax.ShapeDtypeStruct((n, c), jnp.float32),
    )(p1, p1, g1, degp1, degp1, b1, w2, degp2, degp2)


def _tc_x2(p2, g2, degp2, b2):
    n, c = g2.shape

    def body(p0_ref, p1_ref, g_ref, d0_ref, d1_ref, b_ref, x2_ref):
        deg = d0_ref[0, :, 0] + d1_ref[0, :, 0] + 1.0
        s = p0_ref[0] + p1_ref[0] + g_ref[...]
        x2_ref[...] = s * lax.rsqrt(deg)[:, None] + b_ref[...]

    return pl.pallas_call(
        body,
        grid=(n // BM,),
        in_specs=[pl.BlockSpec((1, BM, c), lambda i: (0, i, 0)),
                  pl.BlockSpec((1, BM, c), lambda i: (1, i, 0)),
                  pl.BlockSpec((BM, c), lambda i: (i, 0)),
                  pl.BlockSpec((1, BM, 16), lambda i: (0, i, 0)),
                  pl.BlockSpec((1, BM, 16), lambda i: (1, i, 0)),
                  pl.BlockSpec((1, c), lambda i: (0, 0))],
        out_specs=pl.BlockSpec((BM, c), lambda i: (i, 0)),
        out_shape=jax.ShapeDtypeStruct((n, c), jnp.float32),
    )(p2, p2, g2, degp2, degp2, b2)


def _tc_log_softmax(x2):
    n, c = x2.shape

    def body(x_ref, lp_ref):
        x = x_ref[...]
        m = jnp.max(x, axis=1, keepdims=True)
        e = jnp.exp(x - m)
        lp_ref[...] = x - m - jnp.log(jnp.sum(e, axis=1, keepdims=True))

    return pl.pallas_call(
        body,
        grid=(n // BM,),
        in_specs=[pl.BlockSpec((BM, c), lambda i: (i, 0))],
        out_specs=pl.BlockSpec((BM, c), lambda i: (i, 0)),
        out_shape=jax.ShapeDtypeStruct((n, c), jnp.float32),
    )(x2)


def kernel(data, pos_edge_index, neg_edge_index, edge_index, W1, b1, W2, b2):
    f32 = jnp.float32
    n, _ = data.shape
    h = W1.shape[1]
    c = W2.shape[1]
    np_ = -(-n // (NS * 8)) * (NS * 8)
    if np_ - n < 16:
        np_ += NS * 8

    ei = edge_index.astype(jnp.int32)
    pe = pos_edge_index.astype(jnp.int32)
    ne = neg_edge_index.astype(jnp.int32)
    el = pe.shape[1] + ne.shape[1]

    s1, d1, _ = _pad_edges(ei[0], ei[1], n)
    s2, d2, _ = _pad_edges(pe[0], pe[1], n)
    lj, li, elp = _pad_edges(jnp.concatenate([pe[0], ne[0]]),
                             jnp.concatenate([pe[1], ne[1]]), n)

    zeros = jnp.zeros((np_, 128), f32)
    zeros16 = jnp.zeros((np_, 16), f32)
    zeros64 = jnp.zeros((np_, c), f32)
    degp1_flat, degp2_flat = _sc_degrees(d1, d2, np_, zeros16)
    degp1 = degp1_flat.reshape(NC, np_, 16)
    degp2 = degp2_flat.reshape(NC, np_, 16)

    g1 = _tc_matmul_scale(data, W1, degp1)

    pad1 = jnp.zeros((np_ - n, h), f32)
    p1 = _sc_segment_sum(jnp.concatenate([g1, pad1]), s1, d1, np_, h,
                         zeros, 4, False, 64)
    p1 = p1.reshape(NC, np_, h)

    g2 = _tc_layer2(p1, g1, degp1, b1.reshape(1, h), W2, degp2)

    p2 = _sc_segment_sum(jnp.concatenate([g2, zeros64[: np_ - n]]), s2, d2,
                         np_, c, zeros64, 4, True)
    p2 = p2.reshape(NC, np_, c)

    x2 = _tc_x2(p2, g2, degp2, b2.reshape(1, c))

    x2p = jnp.concatenate([x2, jnp.zeros((np_ - n, c), f32)])
    link_logits = _sc_link_logits(x2p, lj, li, c, c)[:el]
    log_probs = _tc_log_softmax(x2)

    return log_probs, link_logits


# speedup vs baseline: 1.0538x; 1.0538x over previous
import functools

import jax
import jax.numpy as jnp
from jax import lax
from jax.experimental import pallas as pl
from jax.experimental.pallas import tpu as pltpu
from jax.experimental.pallas import tpu_sc as plsc

NC, NS = 2, 16
NW = NC * NS
CH = 128
UNIT = NW * CH
BM = 400
BE = 2048
U = 4


def _sc_mesh():
    return plsc.VectorSubcoreMesh(core_axis_name="c", subcore_axis_name="s",
                                  num_cores=NC, num_subcores=NS)


def _pad_edges(src, dst, n):
    e = src.shape[0]
    ep = -(-e // UNIT) * UNIT
    if ep != e:
        fill = n + (jnp.arange(ep - e, dtype=jnp.int32) % 16)
        src = jnp.concatenate([src, fill])
        dst = jnp.concatenate([dst, fill])
    return src, dst, ep


def _sc_degrees(dst1, dst2, np_, zeros):
    nch1 = dst1.shape[0] // UNIT
    nch2 = dst2.shape[0] // UNIT
    rp = np_ // NS
    ones = jnp.ones((CH, 16), jnp.float32)
    d1r = dst1.reshape(-1, CH)
    d2r = dst2.reshape(-1, CH)

    @functools.partial(
        pl.kernel,
        out_type=(jax.ShapeDtypeStruct((NC * np_, 16), jnp.float32),
                  jax.ShapeDtypeStruct((NC * np_, 16), jnp.float32)),
        mesh=_sc_mesh(),
        compiler_params=pltpu.CompilerParams(use_tc_tiling_on_sc=False),
        scratch_types=[
            pltpu.VMEM((U, CH), jnp.int32),
            pltpu.VMEM((CH, 16), jnp.float32),
            pltpu.VMEM_SHARED((np_, 16), jnp.float32),
        ] + [pltpu.SemaphoreType.DMA] * U,
    )
    def k(d1_hbm, d2_hbm, ones_hbm, z_hbm, o1_hbm, o2_hbm,
          idx_v, ones_v, acc, *sems):
        cid = lax.axis_index("c")
        sid = lax.axis_index("s")
        wid = cid * NS + sid
        row = cid * np_ + sid * rp
        pltpu.sync_copy(ones_hbm, ones_v)

        def one_set(d_hbm, nch, o_hbm):
            pltpu.sync_copy(z_hbm.at[pl.ds(sid * rp, rp)],
                            acc.at[pl.ds(sid * rp, rp)])
            plsc.subcore_barrier()
            base = wid * nch

            @pl.loop(0, nch // U)
            def _(j):
                pltpu.sync_copy(d_hbm.at[pl.ds(base + j * U, U)], idx_v)
                ds_ = [pltpu.async_copy(ones_v, acc.at[idx_v.at[b]], sems[b],
                                        add=True) for b in range(U)]
                for d_ in ds_:
                    d_.wait()

            plsc.subcore_barrier()
            pltpu.sync_copy(acc.at[pl.ds(sid * rp, rp)], o_hbm.at[pl.ds(row, rp)])

        one_set(d1_hbm, nch1, o1_hbm)
        plsc.subcore_barrier()
        one_set(d2_hbm, nch2, o2_hbm)

    return k(d1r, d2r, ones, zeros)


def _sc_segment_sum(g_pad, src, dst, np_, d, zeros, us, linear, chp=CH):
    nch = src.shape[0] // (NW * chp)
    rp = np_ // NS
    s2d = src.reshape(-1, chp)
    d2d = dst.reshape(-1, chp)
    US = us

    @functools.partial(
        pl.kernel,
        out_type=jax.ShapeDtypeStruct((NC * np_, d), jnp.float32),
        mesh=_sc_mesh(),
        compiler_params=(pltpu.CompilerParams(use_tc_tiling_on_sc=False)
                         if linear else None),
        scratch_types=[
            pltpu.VMEM((US, chp), jnp.int32),
            pltpu.VMEM((US, chp), jnp.int32),
        ] + [pltpu.VMEM((chp, d), jnp.float32)] * US + [
            pltpu.VMEM_SHARED((np_, d), jnp.float32),
        ] + [pltpu.SemaphoreType.DMA] * (2 * US),
    )
    def k(g_hbm, s_hbm, d_hbm, z_hbm, o_hbm, sidx, didx, *rest):
        rows = rest[:US]
        acc = rest[US]
        gsem = rest[US + 1:US + 1 + US]
        ssem = rest[US + 1 + US:]
        cid = lax.axis_index("c")
        sid = lax.axis_index("s")
        wid = cid * NS + sid
        pltpu.sync_copy(z_hbm.at[pl.ds(sid * rp, rp)], acc.at[pl.ds(sid * rp, rp)])
        plsc.subcore_barrier()
        base = wid * nch

        @pl.loop(0, nch // US)
        def _(j):
            row0 = base + j * US
            pltpu.sync_copy(s_hbm.at[pl.ds(row0, US)], sidx)
            pltpu.sync_copy(d_hbm.at[pl.ds(row0, US)], didx)
            gd = [pltpu.async_copy(g_hbm.at[sidx.at[b]], rows[b], gsem[b])
                  for b in range(US)]
            sd = []
            for b in range(US):
                gd[b].wait()
                sd.append(pltpu.async_copy(rows[b], acc.at[didx.at[b]],
                                           ssem[b], add=True))
            for d_ in sd:
                d_.wait()

        plsc.subcore_barrier()
        row = cid * np_ + sid * rp
        pltpu.sync_copy(acc.at[pl.ds(sid * rp, rp)], o_hbm.at[pl.ds(row, rp)])

    return k(g_pad, s2d, d2d, zeros)


def _sc_link_logits(x_pad, idx_j, idx_i, d, real_c):
    elp = idx_j.shape[0]
    nch = elp // UNIT
    UL = 4
    nv = real_c // 16
    j2d = idx_j.reshape(-1, CH)
    i2d = idx_i.reshape(-1, CH)

    @functools.partial(
        pl.kernel,
        out_type=jax.ShapeDtypeStruct((elp,), jnp.float32),
        mesh=_sc_mesh(),
        compiler_params=pltpu.CompilerParams(needs_layout_passes=False,
                                             use_tc_tiling_on_sc=False),
        scratch_types=[
            pltpu.VMEM((UL, CH), jnp.int32),
            pltpu.VMEM((UL, CH), jnp.int32),
        ] + [pltpu.VMEM((CH, d), jnp.float32)] * (2 * UL)
          + [pltpu.VMEM((CH,), jnp.float32)] * UL
          + [pltpu.SemaphoreType.DMA] * (3 * UL),
    )
    def k(x_hbm, j_hbm, i_hbm, o_hbm, jidx, iidx, *rest):
        jrows = rest[:UL]
        irows = rest[UL:2 * UL]
        outv = rest[2 * UL:3 * UL]
        sems = rest[3 * UL:]
        cid = lax.axis_index("c")
        sid = lax.axis_index("s")
        wid = cid * NS + sid
        base = wid * nch

        @pl.loop(0, nch // UL)
        def _(jj):
            row0 = base + jj * UL
            pltpu.sync_copy(j_hbm.at[pl.ds(row0, UL)], jidx)
            pltpu.sync_copy(i_hbm.at[pl.ds(row0, UL)], iidx)
            gd = []
            for b in range(UL):
                gd.append(pltpu.async_copy(x_hbm.at[jidx.at[b]], jrows[b],
                                           sems[2 * b]))
                gd.append(pltpu.async_copy(x_hbm.at[iidx.at[b]], irows[b],
                                           sems[2 * b + 1]))
            wd = []
            for b in range(UL):
                gd[2 * b].wait()
                gd[2 * b + 1].wait()

                def dot_group(grp, b=b):
                    lane = lax.iota(jnp.int32, 16)
                    vec = jnp.zeros((16,), jnp.float32)
                    for rr in range(16):
                        r = grp * 16 + rr
                        acc = (jrows[b][r, pl.ds(0, 16)] *
                               irows[b][r, pl.ds(0, 16)])
                        for kk in range(1, nv):
                            acc = acc + (jrows[b][r, pl.ds(16 * kk, 16)] *
                                         irows[b][r, pl.ds(16 * kk, 16)])
                        vec = jnp.where(lane == rr, jnp.sum(acc), vec)
                    outv[b][pl.ds(grp * 16, 16)] = vec

                pl.loop(0, CH // 16)(dot_group)
                wd.append(pltpu.async_copy(outv[b],
                                           o_hbm.at[pl.ds((row0 + b) * CH, CH)],
                                           sems[2 * UL + b]))
            for d_ in wd:
                d_.wait()

    return k(x_pad, j2d, i2d)


def _tc_matmul_scale(x, w, degp):
    n, kdim = x.shape
    m = w.shape[1]

    def body(x_ref, w_ref, d0_ref, d1_ref, o_ref):
        h = jnp.dot(x_ref[...], w_ref[...], preferred_element_type=jnp.float32)
        deg = d0_ref[0, :, 0] + d1_ref[0, :, 0] + 1.0
        o_ref[...] = h * lax.rsqrt(deg)[:, None]

    return pl.pallas_call(
        body,
        grid=(n // BM,),
        in_specs=[pl.BlockSpec((BM, kdim), lambda i: (i, 0)),
                  pl.BlockSpec((kdim, m), lambda i: (0, 0)),
                  pl.BlockSpec((1, BM, 16), lambda i: (0, i, 0)),
                  pl.BlockSpec((1, BM, 16), lambda i: (1, i, 0))],
        out_specs=pl.BlockSpec((BM, m), lambda i: (i, 0)),
        out_shape=jax.ShapeDtypeStruct((n, m), jnp.float32),
    )(x, w, degp, degp)


def _tc_layer2(p1, g1, degp1, b1, w2, degp2):
    n, h = g1.shape
    c = w2.shape[1]
    np_ = degp1.shape[1]

    def body(p0_ref, p1_ref, g_ref, d10_ref, d11_ref, b_ref, w_ref,
             d20_ref, d21_ref, o_ref):
        deg1 = d10_ref[0, :, 0] + d11_ref[0, :, 0] + 1.0
        s = p0_ref[0] + p1_ref[0] + g_ref[...]
        x1 = jnp.maximum(s * lax.rsqrt(deg1)[:, None] + b_ref[...], 0.0)
        h2 = jnp.dot(x1, w_ref[...], preferred_element_type=jnp.float32)
        deg2 = d20_ref[0, :, 0] + d21_ref[0, :, 0] + 1.0
        o_ref[...] = h2 * lax.rsqrt(deg2)[:, None]

    return pl.pallas_call(
        body,
        grid=(n // BM,),
        in_specs=[pl.BlockSpec((1, BM, h), lambda i: (0, i, 0)),
                  pl.BlockSpec((1, BM, h), lambda i: (1, i, 0)),
                  pl.BlockSpec((BM, h), lambda i: (i, 0)),
                  pl.BlockSpec((1, BM, 16), lambda i: (0, i, 0)),
                  pl.BlockSpec((1, BM, 16), lambda i: (1, i, 0)),
                  pl.BlockSpec((1, h), lambda i: (0, 0)),
                  pl.BlockSpec((h, c), lambda i: (0, 0)),
                  pl.BlockSpec((1, BM, 16), lambda i: (0, i, 0)),
                  pl.BlockSpec((1, BM, 16), lambda i: (1, i, 0))],
        out_specs=pl.BlockSpec((BM, c), lambda i: (i, 0)),
        out_shape=jax.ShapeDtypeStruct((n, c), jnp.float32),
    )(p1, p1, g1, degp1, degp1, b1, w2, degp2, degp2)


def _tc_x2(p2, g2, degp2, b2):
    n, c = g2.shape

    def body(p0_ref, p1_ref, g_ref, d0_ref, d1_ref, b_ref, x2_ref):
        deg = d0_ref[0, :, 0] + d1_ref[0, :, 0] + 1.0
        s = p0_ref[0] + p1_ref[0] + g_ref[...]
        x2_ref[...] = s * lax.rsqrt(deg)[:, None] + b_ref[...]

    return pl.pallas_call(
        body,
        grid=(n // BM,),
        in_specs=[pl.BlockSpec((1, BM, c), lambda i: (0, i, 0)),
                  pl.BlockSpec((1, BM, c), lambda i: (1, i, 0)),
                  pl.BlockSpec((BM, c), lambda i: (i, 0)),
                  pl.BlockSpec((1, BM, 16), lambda i: (0, i, 0)),
                  pl.BlockSpec((1, BM, 16), lambda i: (1, i, 0)),
                  pl.BlockSpec((1, c), lambda i: (0, 0))],
        out_specs=pl.BlockSpec((BM, c), lambda i: (i, 0)),
        out_shape=jax.ShapeDtypeStruct((n, c), jnp.float32),
    )(p2, p2, g2, degp2, degp2, b2)


def _tc_log_softmax(x2):
    n, c = x2.shape

    def body(x_ref, lp_ref):
        x = x_ref[...]
        m = jnp.max(x, axis=1, keepdims=True)
        e = jnp.exp(x - m)
        lp_ref[...] = x - m - jnp.log(jnp.sum(e, axis=1, keepdims=True))

    return pl.pallas_call(
        body,
        grid=(n // BM,),
        in_specs=[pl.BlockSpec((BM, c), lambda i: (i, 0))],
        out_specs=pl.BlockSpec((BM, c), lambda i: (i, 0)),
        out_shape=jax.ShapeDtypeStruct((n, c), jnp.float32),
    )(x2)


def kernel(data, pos_edge_index, neg_edge_index, edge_index, W1, b1, W2, b2):
    f32 = jnp.float32
    n, _ = data.shape
    h = W1.shape[1]
    c = W2.shape[1]
    np_ = -(-n // (NS * 8)) * (NS * 8)
    if np_ - n < 16:
        np_ += NS * 8

    ei = edge_index.astype(jnp.int32)
    pe = pos_edge_index.astype(jnp.int32)
    ne = neg_edge_index.astype(jnp.int32)
    el = pe.shape[1] + ne.shape[1]

    s1, d1, _ = _pad_edges(ei[0], ei[1], n)
    s2, d2, _ = _pad_edges(pe[0], pe[1], n)
    lj, li, elp = _pad_edges(jnp.concatenate([pe[0], ne[0]]),
                             jnp.concatenate([pe[1], ne[1]]), n)

    zeros = jnp.zeros((np_, 128), f32)
    zeros16 = jnp.zeros((np_, 16), f32)
    zeros64 = jnp.zeros((np_, c), f32)
    degp1_flat, degp2_flat = _sc_degrees(d1, d2, np_, zeros16)
    degp1 = degp1_flat.reshape(NC, np_, 16)
    degp2 = degp2_flat.reshape(NC, np_, 16)

    g1 = _tc_matmul_scale(data, W1, degp1)

    pad1 = jnp.zeros((np_ - n, h), f32)
    p1 = _sc_segment_sum(jnp.concatenate([g1, pad1]), s1, d1, np_, h,
                         zeros, 2, False)
    p1 = p1.reshape(NC, np_, h)

    g2 = _tc_layer2(p1, g1, degp1, b1.reshape(1, h), W2, degp2)

    p2 = _sc_segment_sum(jnp.concatenate([g2, zeros64[: np_ - n]]), s2, d2,
                         np_, c, zeros64, 4, True)
    p2 = p2.reshape(NC, np_, c)

    x2 = _tc_x2(p2, g2, degp2, b2.reshape(1, c))

    x2p = jnp.concatenate([x2, jnp.zeros((np_ - n, c), f32)])
    link_logits = _sc_link_logits(x2p, lj, li, c, c)[:el]
    log_probs = _tc_log_softmax(x2)

    return log_probs, link_logits


# speedup vs baseline: 1.0614x; 1.0072x over previous
import functools

import jax
import jax.numpy as jnp
from jax import lax
from jax.experimental import pallas as pl
from jax.experimental.pallas import tpu as pltpu
from jax.experimental.pallas import tpu_sc as plsc

NC, NS = 2, 16
NW = NC * NS
CH = 128
UNIT = NW * CH
BM = 400
BE = 2048
U = 4


def _sc_mesh():
    return plsc.VectorSubcoreMesh(core_axis_name="c", subcore_axis_name="s",
                                  num_cores=NC, num_subcores=NS)


def _pad_edges(src, dst, n):
    e = src.shape[0]
    ep = -(-e // UNIT) * UNIT
    if ep != e:
        fill = n + (jnp.arange(ep - e, dtype=jnp.int32) % 16)
        src = jnp.concatenate([src, fill])
        dst = jnp.concatenate([dst, fill])
    return src, dst, ep


def _sc_degrees(dst1, dst2, np_, zeros):
    nch1 = dst1.shape[0] // UNIT
    nch2 = dst2.shape[0] // UNIT
    rp = np_ // NS
    ones = jnp.ones((CH, 16), jnp.float32)
    d1r = dst1.reshape(-1, CH)
    d2r = dst2.reshape(-1, CH)

    @functools.partial(
        pl.kernel,
        out_type=(jax.ShapeDtypeStruct((NC * np_, 16), jnp.float32),
                  jax.ShapeDtypeStruct((NC * np_, 16), jnp.float32)),
        mesh=_sc_mesh(),
        compiler_params=pltpu.CompilerParams(use_tc_tiling_on_sc=False),
        scratch_types=[
            pltpu.VMEM((U, CH), jnp.int32),
            pltpu.VMEM((CH, 16), jnp.float32),
            pltpu.VMEM_SHARED((np_, 16), jnp.float32),
        ] + [pltpu.SemaphoreType.DMA] * U,
    )
    def k(d1_hbm, d2_hbm, ones_hbm, z_hbm, o1_hbm, o2_hbm,
          idx_v, ones_v, acc, *sems):
        cid = lax.axis_index("c")
        sid = lax.axis_index("s")
        wid = cid * NS + sid
        row = cid * np_ + sid * rp
        pltpu.sync_copy(ones_hbm, ones_v)

        def one_set(d_hbm, nch, o_hbm):
            pltpu.sync_copy(z_hbm.at[pl.ds(sid * rp, rp)],
                            acc.at[pl.ds(sid * rp, rp)])
            plsc.subcore_barrier()
            base = wid * nch

            @pl.loop(0, nch // U)
            def _(j):
                pltpu.sync_copy(d_hbm.at[pl.ds(base + j * U, U)], idx_v)
                ds_ = [pltpu.async_copy(ones_v, acc.at[idx_v.at[b]], sems[b],
                                        add=True) for b in range(U)]
                for d_ in ds_:
                    d_.wait()

            plsc.subcore_barrier()
            pltpu.sync_copy(acc.at[pl.ds(sid * rp, rp)], o_hbm.at[pl.ds(row, rp)])

        one_set(d1_hbm, nch1, o1_hbm)
        plsc.subcore_barrier()
        one_set(d2_hbm, nch2, o2_hbm)

    return k(d1r, d2r, ones, zeros)


def _sc_segment_sum(g_pad, src, dst, np_, d, zeros, us, linear, chp=CH):
    nch = src.shape[0] // (NW * chp)
    rp = np_ // NS
    s2d = src.reshape(-1, chp)
    d2d = dst.reshape(-1, chp)
    US = us

    @functools.partial(
        pl.kernel,
        out_type=jax.ShapeDtypeStruct((NC * np_, d), jnp.float32),
        mesh=_sc_mesh(),
        compiler_params=(pltpu.CompilerParams(use_tc_tiling_on_sc=False)
                         if linear else None),
        scratch_types=[
            pltpu.VMEM((US, chp), jnp.int32),
            pltpu.VMEM((US, chp), jnp.int32),
        ] + [pltpu.VMEM((chp, d), jnp.float32)] * US + [
            pltpu.VMEM_SHARED((np_, d), jnp.float32),
        ] + [pltpu.SemaphoreType.DMA] * (2 * US),
    )
    def k(g_hbm, s_hbm, d_hbm, z_hbm, o_hbm, sidx, didx, *rest):
        rows = rest[:US]
        acc = rest[US]
        gsem = rest[US + 1:US + 1 + US]
        ssem = rest[US + 1 + US:]
        cid = lax.axis_index("c")
        sid = lax.axis_index("s")
        wid = cid * NS + sid
        pltpu.sync_copy(z_hbm.at[pl.ds(sid * rp, rp)], acc.at[pl.ds(sid * rp, rp)])
        plsc.subcore_barrier()
        base = wid * nch

        @pl.loop(0, nch // US)
        def _(j):
            row0 = base + j * US
            pltpu.sync_copy(s_hbm.at[pl.ds(row0, US)], sidx)
            pltpu.sync_copy(d_hbm.at[pl.ds(row0, US)], didx)
            gd = [pltpu.async_copy(g_hbm.at[sidx.at[b]], rows[b], gsem[b])
                  for b in range(US)]
            sd = []
            for b in range(US):
                gd[b].wait()
                sd.append(pltpu.async_copy(rows[b], acc.at[didx.at[b]],
                                           ssem[b], add=True))
            for d_ in sd:
                d_.wait()

        plsc.subcore_barrier()
        row = cid * np_ + sid * rp
        pltpu.sync_copy(acc.at[pl.ds(sid * rp, rp)], o_hbm.at[pl.ds(row, rp)])

    return k(g_pad, s2d, d2d, zeros)


def _sc_link_logits(x_pad, idx_j, idx_i, d, real_c):
    elp = idx_j.shape[0]
    nch = elp // UNIT
    UL = 5
    nv = real_c // 16
    j2d = idx_j.reshape(-1, CH)
    i2d = idx_i.reshape(-1, CH)

    @functools.partial(
        pl.kernel,
        out_type=jax.ShapeDtypeStruct((elp,), jnp.float32),
        mesh=_sc_mesh(),
        compiler_params=pltpu.CompilerParams(needs_layout_passes=False,
                                             use_tc_tiling_on_sc=False),
        scratch_types=[
            pltpu.VMEM((UL, CH), jnp.int32),
            pltpu.VMEM((UL, CH), jnp.int32),
        ] + [pltpu.VMEM((CH, d), jnp.float32)] * (2 * UL)
          + [pltpu.VMEM((CH,), jnp.float32)] * UL
          + [pltpu.SemaphoreType.DMA] * (3 * UL),
    )
    def k(x_hbm, j_hbm, i_hbm, o_hbm, jidx, iidx, *rest):
        jrows = rest[:UL]
        irows = rest[UL:2 * UL]
        outv = rest[2 * UL:3 * UL]
        sems = rest[3 * UL:]
        cid = lax.axis_index("c")
        sid = lax.axis_index("s")
        wid = cid * NS + sid
        base = wid * nch

        @pl.loop(0, nch // UL)
        def _(jj):
            row0 = base + jj * UL
            pltpu.sync_copy(j_hbm.at[pl.ds(row0, UL)], jidx)
            pltpu.sync_copy(i_hbm.at[pl.ds(row0, UL)], iidx)
            gd = []
            for b in range(UL):
                gd.append(pltpu.async_copy(x_hbm.at[jidx.at[b]], jrows[b],
                                           sems[2 * b]))
                gd.append(pltpu.async_copy(x_hbm.at[iidx.at[b]], irows[b],
                                           sems[2 * b + 1]))
            wd = []
            for b in range(UL):
                gd[2 * b].wait()
                gd[2 * b + 1].wait()

                def dot_group(grp, b=b):
                    lane = lax.iota(jnp.int32, 16)
                    vec = jnp.zeros((16,), jnp.float32)
                    for rr in range(16):
                        r = grp * 16 + rr
                        acc = (jrows[b][r, pl.ds(0, 16)] *
                               irows[b][r, pl.ds(0, 16)])
                        for kk in range(1, nv):
                            acc = acc + (jrows[b][r, pl.ds(16 * kk, 16)] *
                                         irows[b][r, pl.ds(16 * kk, 16)])
                        vec = jnp.where(lane == rr, jnp.sum(acc), vec)
                    outv[b][pl.ds(grp * 16, 16)] = vec

                pl.loop(0, CH // 16)(dot_group)
                wd.append(pltpu.async_copy(outv[b],
                                           o_hbm.at[pl.ds((row0 + b) * CH, CH)],
                                           sems[2 * UL + b]))
            for d_ in wd:
                d_.wait()

    return k(x_pad, j2d, i2d)


def _tc_matmul_scale(x, w, degp):
    n, kdim = x.shape
    m = w.shape[1]

    def body(x_ref, w_ref, d0_ref, d1_ref, o_ref):
        h = jnp.dot(x_ref[...], w_ref[...], preferred_element_type=jnp.float32)
        deg = d0_ref[0, :, 0] + d1_ref[0, :, 0] + 1.0
        o_ref[...] = h * lax.rsqrt(deg)[:, None]

    return pl.pallas_call(
        body,
        grid=(n // BM,),
        in_specs=[pl.BlockSpec((BM, kdim), lambda i: (i, 0)),
                  pl.BlockSpec((kdim, m), lambda i: (0, 0)),
                  pl.BlockSpec((1, BM, 16), lambda i: (0, i, 0)),
                  pl.BlockSpec((1, BM, 16), lambda i: (1, i, 0))],
        out_specs=pl.BlockSpec((BM, m), lambda i: (i, 0)),
        out_shape=jax.ShapeDtypeStruct((n, m), jnp.float32),
    )(x, w, degp, degp)


def _tc_layer2(p1, g1, degp1, b1, w2, degp2):
    n, h = g1.shape
    c = w2.shape[1]
    np_ = degp1.shape[1]

    def body(p0_ref, p1_ref, g_ref, d10_ref, d11_ref, b_ref, w_ref,
             d20_ref, d21_ref, o_ref):
        deg1 = d10_ref[0, :, 0] + d11_ref[0, :, 0] + 1.0
        s = p0_ref[0] + p1_ref[0] + g_ref[...]
        x1 = jnp.maximum(s * lax.rsqrt(deg1)[:, None] + b_ref[...], 0.0)
        h2 = jnp.dot(x1, w_ref[...], preferred_element_type=jnp.float32)
        deg2 = d20_ref[0, :, 0] + d21_ref[0, :, 0] + 1.0
        o_ref[...] = h2 * lax.rsqrt(deg2)[:, None]

    return pl.pallas_call(
        body,
        grid=(n // BM,),
        in_specs=[pl.BlockSpec((1, BM, h), lambda i: (0, i, 0)),
                  pl.BlockSpec((1, BM, h), lambda i: (1, i, 0)),
                  pl.BlockSpec((BM, h), lambda i: (i, 0)),
                  pl.BlockSpec((1, BM, 16), lambda i: (0, i, 0)),
                  pl.BlockSpec((1, BM, 16), lambda i: (1, i, 0)),
                  pl.BlockSpec((1, h), lambda i: (0, 0)),
                  pl.BlockSpec((h, c), lambda i: (0, 0)),
                  pl.BlockSpec((1, BM, 16), lambda i: (0, i, 0)),
                  pl.BlockSpec((1, BM, 16), lambda i: (1, i, 0))],
        out_specs=pl.BlockSpec((BM, c), lambda i: (i, 0)),
        out_shape=jax.ShapeDtypeStruct((n, c), jnp.float32),
    )(p1, p1, g1, degp1, degp1, b1, w2, degp2, degp2)


def _tc_x2(p2, g2, degp2, b2):
    n, c = g2.shape

    def body(p0_ref, p1_ref, g_ref, d0_ref, d1_ref, b_ref, x2_ref):
        deg = d0_ref[0, :, 0] + d1_ref[0, :, 0] + 1.0
        s = p0_ref[0] + p1_ref[0] + g_ref[...]
        x2_ref[...] = s * lax.rsqrt(deg)[:, None] + b_ref[...]

    return pl.pallas_call(
        body,
        grid=(n // BM,),
        in_specs=[pl.BlockSpec((1, BM, c), lambda i: (0, i, 0)),
                  pl.BlockSpec((1, BM, c), lambda i: (1, i, 0)),
                  pl.BlockSpec((BM, c), lambda i: (i, 0)),
                  pl.BlockSpec((1, BM, 16), lambda i: (0, i, 0)),
                  pl.BlockSpec((1, BM, 16), lambda i: (1, i, 0)),
                  pl.BlockSpec((1, c), lambda i: (0, 0))],
        out_specs=pl.BlockSpec((BM, c), lambda i: (i, 0)),
        out_shape=jax.ShapeDtypeStruct((n, c), jnp.float32),
    )(p2, p2, g2, degp2, degp2, b2)


def _tc_log_softmax(x2):
    n, c = x2.shape

    def body(x_ref, lp_ref):
        x = x_ref[...]
        m = jnp.max(x, axis=1, keepdims=True)
        e = jnp.exp(x - m)
        lp_ref[...] = x - m - jnp.log(jnp.sum(e, axis=1, keepdims=True))

    return pl.pallas_call(
        body,
        grid=(n // BM,),
        in_specs=[pl.BlockSpec((BM, c), lambda i: (i, 0))],
        out_specs=pl.BlockSpec((BM, c), lambda i: (i, 0)),
        out_shape=jax.ShapeDtypeStruct((n, c), jnp.float32),
    )(x2)


def kernel(data, pos_edge_index, neg_edge_index, edge_index, W1, b1, W2, b2):
    f32 = jnp.float32
    n, _ = data.shape
    h = W1.shape[1]
    c = W2.shape[1]
    np_ = -(-n // (NS * 8)) * (NS * 8)
    if np_ - n < 16:
        np_ += NS * 8

    ei = edge_index.astype(jnp.int32)
    pe = pos_edge_index.astype(jnp.int32)
    ne = neg_edge_index.astype(jnp.int32)
    el = pe.shape[1] + ne.shape[1]

    s1, d1, _ = _pad_edges(ei[0], ei[1], n)
    s2, d2, _ = _pad_edges(pe[0], pe[1], n)
    lj, li, elp = _pad_edges(jnp.concatenate([pe[0], ne[0]]),
                             jnp.concatenate([pe[1], ne[1]]), n)

    zeros = jnp.zeros((np_, 128), f32)
    zeros16 = jnp.zeros((np_, 16), f32)
    zeros64 = jnp.zeros((np_, c), f32)
    degp1_flat, degp2_flat = _sc_degrees(d1, d2, np_, zeros16)
    degp1 = degp1_flat.reshape(NC, np_, 16)
    degp2 = degp2_flat.reshape(NC, np_, 16)

    g1 = _tc_matmul_scale(data, W1, degp1)

    pad1 = jnp.zeros((np_ - n, h), f32)
    p1 = _sc_segment_sum(jnp.concatenate([g1, pad1]), s1, d1, np_, h,
                         zeros, 2, False)
    p1 = p1.reshape(NC, np_, h)

    g2 = _tc_layer2(p1, g1, degp1, b1.reshape(1, h), W2, degp2)

    p2 = _sc_segment_sum(jnp.concatenate([g2, zeros64[: np_ - n]]), s2, d2,
                         np_, c, zeros64, 4, True)
    p2 = p2.reshape(NC, np_, c)

    x2 = _tc_x2(p2, g2, degp2, b2.reshape(1, c))

    x2p = jnp.concatenate([x2, jnp.zeros((np_ - n, c), f32)])
    link_logits = _sc_link_logits(x2p, lj, li, c, c)[:el]
    log_probs = _tc_log_softmax(x2)

    return log_probs, link_logits
